# baseline (speedup 1.0000x reference)
import jax
import jax.numpy as jnp
from jax import lax
from jax.experimental import pallas as pl
from jax.experimental.pallas import tpu as pltpu
from jax.experimental.pallas import tpu_sc as plsc

B, S, H, D, K = 64, 32, 256, 64, 8
L = 16
NC, NS = 2, 16
NW = NC * NS
P = B * K
PPW = P // NW
SUB = (H * D) // L
VROWS = B * S * L


def _gather_body(mem_hbm, tk_hbm, out_hbm, tk_v, fl_v, buf_a, buf_b, sem_a, sem_b):
    wid = lax.axis_index("s") * NC + lax.axis_index("c")
    base = wid * PPW

    pltpu.sync_copy(tk_hbm.at[pl.ds(base, PPW)], tk_v)
    lane = lax.broadcasted_iota(jnp.int32, (L,), 0)
    p_vec = base + lane
    b_vec = lax.shift_right_logical(p_vec, 3)
    flat_reg = (b_vec * S + tk_v[...]) * L
    del fl_v

    bufs = (buf_a, buf_b)
    sems = (sem_a, sem_b)

    def issue(i, slot):
        fvec = flat_reg.at[jnp.full((L,), i, jnp.int32)].get(
            mode="promise_in_bounds")
        return pltpu.async_copy(mem_hbm.at[fvec + lane], bufs[slot], sems[slot])

    cp = issue(0, 0)
    for i in range(PPW):
        slot = i % 2
        cp.wait()
        if i + 1 < PPW:
            cp = issue(i + 1, (i + 1) % 2)
        pltpu.sync_copy(bufs[slot], out_hbm.at[base + i])


def kernel(memories_per_batch, topk_indices, gates):
    del gates
    mem2 = memories_per_batch.reshape(VROWS, SUB)
    tk = topk_indices.reshape(P).astype(jnp.int32)

    mesh = plsc.VectorSubcoreMesh(
        core_axis_name="c", subcore_axis_name="s", num_cores=NC, num_subcores=NS
    )
    out = pl.kernel(
        _gather_body,
        out_type=jax.ShapeDtypeStruct((P, L, SUB), jnp.float32),
        mesh=mesh,
        scratch_types=[
            pltpu.VMEM((PPW,), jnp.int32),
            pltpu.VMEM((L,), jnp.int32),
            pltpu.VMEM((L, SUB), jnp.float32),
            pltpu.VMEM((L, SUB), jnp.float32),
            pltpu.SemaphoreType.DMA,
            pltpu.SemaphoreType.DMA,
        ],
    )(mem2, tk)
    return out.reshape(B, K, H, D)
